# baseline (speedup 1.0000x reference)
import functools

import jax
import jax.numpy as jnp
from jax import lax
from jax.experimental import pallas as pl
from jax.experimental.pallas import tpu as pltpu
from jax.experimental.pallas import tpu_sc as plsc

N, T, C, H, W = 8, 8, 128, 32, 32
F = C * H * W
TO = 16
NC, NS = 2, 16
NW = NC * NS
WPN = NW // N
CH = 4096
ITEMS = F // CH // WPN
UNROLL = 16
LANES = 16
NEG = -1e30


@functools.partial(
    pl.kernel,
    out_type=jax.ShapeDtypeStruct((N * TO * F,), jnp.float32),
    mesh=plsc.VectorSubcoreMesh(core_axis_name="c", subcore_axis_name="s"),
    compiler_params=pltpu.CompilerParams(needs_layout_passes=False),
    scratch_types=[
        pltpu.VMEM((2, T * CH), jnp.float32),
        pltpu.VMEM((4, CH), jnp.float32),
        pltpu.VMEM((2 * LANES,), jnp.float32),
        pltpu.SemaphoreType.DMA,
        pltpu.SemaphoreType.DMA,
        pltpu.SemaphoreType.DMA,
        pltpu.SemaphoreType.DMA,
        pltpu.SemaphoreType.DMA,
        pltpu.SemaphoreType.DMA,
    ],
)
def _interp_sc(u_hbm, idx_hbm, out_hbm, in_v, out_v, idx_v,
               sem_i0, sem_i1, sem_o0, sem_o1, sem_o2, sem_o3):
    cid = lax.axis_index("c")
    sid = lax.axis_index("s")
    wid = sid * NC + cid
    n = wid // WPN
    slot = wid % WPN

    pltpu.sync_copy(idx_hbm.at[pl.ds(n * (2 * LANES), 2 * LANES)], idx_v)
    v1 = idx_v[pl.ds(0, LANES)]
    v2 = idx_v[pl.ds(LANES, LANES)]
    m = jnp.maximum(jnp.max(v1), jnp.max(v2))
    e1 = jnp.exp(v1 - m)
    e2 = jnp.exp(v2 - m)
    tot = jnp.sum(e1) + jnp.sum(e2)
    tf = (plsc.cumsum(e1) * float(T - 1)) / jnp.broadcast_to(tot, (LANES,))
    tl = jnp.minimum(tf.astype(jnp.int32), T - 2)
    alpha = tf - tl.astype(jnp.float32)
    wl = 1.0 - alpha
    wr = alpha
    base_ls = [tl[o] * CH for o in range(TO)]
    wl_ss = [wl[o] for o in range(TO)]
    wr_ss = [wr[o] for o in range(TO)]

    isems = (sem_i0, sem_i1)
    osems = (sem_o0, sem_o1, sem_o2, sem_o3)

    def fire(item, ib):
        cbase = (slot * ITEMS + item) * CH
        for t in range(T):
            pltpu.make_async_copy(
                u_hbm.at[pl.ds((n * T + t) * F + cbase, CH)],
                in_v.at[ib, pl.ds(t * CH, CH)],
                isems[ib]).start()

    def drain_in(ib):
        pltpu.make_async_copy(u_hbm.at[pl.ds(0, T * CH)], in_v.at[ib],
                              isems[ib]).wait()

    def wait_out(b):
        pltpu.make_async_copy(out_v.at[b], out_hbm.at[pl.ds(0, CH)],
                              osems[b]).wait()

    def do_item(item, ib):
        cbase = (slot * ITEMS + item) * CH
        for po in range(0, TO, 2):
            o0, o1 = po, po + 1
            b0, b1 = o0 % 4, o1 % 4

            @pl.when(item * TO + po >= 4)
            def _():
                wait_out(b0)
                wait_out(b1)

            bl0, bl1 = base_ls[o0], base_ls[o1]
            w0l, w0r = wl_ss[o0], wr_ss[o0]
            w1l, w1r = wl_ss[o1], wr_ss[o1]
            shared = bl0 == bl1

            @pl.when(shared)
            def _():
                def body(off):
                    a = in_v[ib, pl.ds(bl0 + off, LANES)]
                    c = in_v[ib, pl.ds(bl0 + CH + off, LANES)]
                    out_v[b0, pl.ds(off, LANES)] = w0l * a + w0r * c
                    out_v[b1, pl.ds(off, LANES)] = w1l * a + w1r * c
                plsc.parallel_loop(0, CH, LANES, unroll=UNROLL)(body)

            @pl.when(jnp.logical_not(shared))
            def _():
                def body(off):
                    a0 = in_v[ib, pl.ds(bl0 + off, LANES)]
                    c0 = in_v[ib, pl.ds(bl0 + CH + off, LANES)]
                    out_v[b0, pl.ds(off, LANES)] = w0l * a0 + w0r * c0
                    a1 = in_v[ib, pl.ds(bl1 + off, LANES)]
                    c1 = in_v[ib, pl.ds(bl1 + CH + off, LANES)]
                    out_v[b1, pl.ds(off, LANES)] = w1l * a1 + w1r * c1
                plsc.parallel_loop(0, CH, LANES, unroll=UNROLL)(body)

            pltpu.make_async_copy(
                out_v.at[b0],
                out_hbm.at[pl.ds((n * TO + o0) * F + cbase, CH)],
                osems[b0]).start()
            pltpu.make_async_copy(
                out_v.at[b1],
                out_hbm.at[pl.ds((n * TO + o1) * F + cbase, CH)],
                osems[b1]).start()

    fire(0, 0)

    def pair(k, carry):
        item_a = 2 * k
        drain_in(0)
        fire(item_a + 1, 1)
        do_item(item_a, 0)
        drain_in(1)

        @pl.when(item_a + 2 < ITEMS)
        def _():
            fire(item_a + 2, 0)

        do_item(item_a + 1, 1)
        return carry

    lax.fori_loop(0, ITEMS // 2, pair, 0)

    for b in range(4):
        wait_out(b)


def kernel(input, index):
    u_flat = jnp.transpose(input, (0, 1, 3, 4, 2)).reshape(-1)
    idx_pad = jnp.pad(index, ((0, 0), (0, 2 * LANES - index.shape[1])),
                      constant_values=NEG).reshape(-1)
    out = _interp_sc(u_flat, idx_pad)
    return jnp.transpose(out.reshape(N, TO, H, W, C), (0, 1, 4, 2, 3))


# speedup vs baseline: 1.0768x; 1.0768x over previous
import functools

import jax
import jax.numpy as jnp
from jax import lax
from jax.experimental import pallas as pl
from jax.experimental.pallas import tpu as pltpu
from jax.experimental.pallas import tpu_sc as plsc

N, T, C, H, W = 8, 8, 128, 32, 32
F = C * H * W
TO = 16
NC, NS = 2, 16
NW = NC * NS
WPN = NW // N
CH = 4096
ITEMS = F // CH // WPN
UNROLL = 8
LANES = 16
NEG = -1e30


@functools.partial(
    pl.kernel,
    out_type=jax.ShapeDtypeStruct((N * TO * F,), jnp.float32),
    mesh=plsc.VectorSubcoreMesh(core_axis_name="c", subcore_axis_name="s"),
    compiler_params=pltpu.CompilerParams(needs_layout_passes=False),
    scratch_types=[
        pltpu.VMEM((2, T * CH), jnp.float32),
        pltpu.VMEM((4, CH), jnp.float32),
        pltpu.VMEM((2 * LANES,), jnp.float32),
        pltpu.SemaphoreType.DMA,
        pltpu.SemaphoreType.DMA,
        pltpu.SemaphoreType.DMA,
        pltpu.SemaphoreType.DMA,
        pltpu.SemaphoreType.DMA,
        pltpu.SemaphoreType.DMA,
    ],
)
def _interp_sc(u_hbm, idx_hbm, out_hbm, in_v, out_v, idx_v,
               sem_i0, sem_i1, sem_o0, sem_o1, sem_o2, sem_o3):
    cid = lax.axis_index("c")
    sid = lax.axis_index("s")
    wid = sid * NC + cid
    n = wid // WPN
    slot = wid % WPN

    pltpu.sync_copy(idx_hbm.at[pl.ds(n * (2 * LANES), 2 * LANES)], idx_v)
    v1 = idx_v[pl.ds(0, LANES)]
    v2 = idx_v[pl.ds(LANES, LANES)]
    m = jnp.maximum(jnp.max(v1), jnp.max(v2))
    e1 = jnp.exp(v1 - m)
    e2 = jnp.exp(v2 - m)
    tot = jnp.sum(e1) + jnp.sum(e2)
    tf = (plsc.cumsum(e1) * float(T - 1)) / jnp.broadcast_to(tot, (LANES,))
    tl = jnp.minimum(tf.astype(jnp.int32), T - 2)
    alpha = tf - tl.astype(jnp.float32)
    wl = 1.0 - alpha
    wr = alpha
    base_ls = [tl[o] * CH for o in range(TO)]
    wl_ss = [wl[o] for o in range(TO)]
    wr_ss = [wr[o] for o in range(TO)]

    isems = (sem_i0, sem_i1)
    osems = (sem_o0, sem_o1, sem_o2, sem_o3)

    def fire(item, ib):
        cbase = (slot * ITEMS + item) * CH
        for t in range(T):
            pltpu.make_async_copy(
                u_hbm.at[pl.ds((n * T + t) * F + cbase, CH)],
                in_v.at[ib, pl.ds(t * CH, CH)],
                isems[ib]).start()

    def drain_in(ib):
        pltpu.make_async_copy(u_hbm.at[pl.ds(0, T * CH)], in_v.at[ib],
                              isems[ib]).wait()

    def wait_out(b):
        pltpu.make_async_copy(out_v.at[b], out_hbm.at[pl.ds(0, CH)],
                              osems[b]).wait()

    def do_item(item, ib):
        cbase = (slot * ITEMS + item) * CH
        for po in range(0, TO, 2):
            o0, o1 = po, po + 1
            b0, b1 = o0 % 4, o1 % 4

            @pl.when(item * TO + po >= 4)
            def _():
                wait_out(b0)
                wait_out(b1)

            bl0, bl1 = base_ls[o0], base_ls[o1]
            w0l, w0r = wl_ss[o0], wr_ss[o0]
            w1l, w1r = wl_ss[o1], wr_ss[o1]
            shared = bl0 == bl1

            @pl.when(shared)
            def _():
                def body(off):
                    a = in_v[ib, pl.ds(bl0 + off, LANES)]
                    c = in_v[ib, pl.ds(bl0 + CH + off, LANES)]
                    out_v[b0, pl.ds(off, LANES)] = w0l * a + w0r * c
                    out_v[b1, pl.ds(off, LANES)] = w1l * a + w1r * c
                plsc.parallel_loop(0, CH, LANES, unroll=UNROLL)(body)

            @pl.when(jnp.logical_not(shared))
            def _():
                def body(off):
                    a0 = in_v[ib, pl.ds(bl0 + off, LANES)]
                    c0 = in_v[ib, pl.ds(bl0 + CH + off, LANES)]
                    out_v[b0, pl.ds(off, LANES)] = w0l * a0 + w0r * c0
                    a1 = in_v[ib, pl.ds(bl1 + off, LANES)]
                    c1 = in_v[ib, pl.ds(bl1 + CH + off, LANES)]
                    out_v[b1, pl.ds(off, LANES)] = w1l * a1 + w1r * c1
                plsc.parallel_loop(0, CH, LANES, unroll=UNROLL)(body)

            pltpu.make_async_copy(
                out_v.at[b0],
                out_hbm.at[pl.ds((n * TO + o0) * F + cbase, CH)],
                osems[b0]).start()
            pltpu.make_async_copy(
                out_v.at[b1],
                out_hbm.at[pl.ds((n * TO + o1) * F + cbase, CH)],
                osems[b1]).start()

    fire(0, 0)

    def pair(k, carry):
        item_a = 2 * k
        drain_in(0)
        fire(item_a + 1, 1)
        do_item(item_a, 0)
        drain_in(1)

        @pl.when(item_a + 2 < ITEMS)
        def _():
            fire(item_a + 2, 0)

        do_item(item_a + 1, 1)
        return carry

    lax.fori_loop(0, ITEMS // 2, pair, 0)

    for b in range(4):
        wait_out(b)


def kernel(input, index):
    u_flat = jnp.transpose(input, (0, 1, 3, 4, 2)).reshape(-1)
    idx_pad = jnp.pad(index, ((0, 0), (0, 2 * LANES - index.shape[1])),
                      constant_values=NEG).reshape(-1)
    out = _interp_sc(u_flat, idx_pad)
    return jnp.transpose(out.reshape(N, TO, H, W, C), (0, 1, 4, 2, 3))


# speedup vs baseline: 1.0955x; 1.0173x over previous
import functools

import jax
import jax.numpy as jnp
from jax import lax
from jax.experimental import pallas as pl
from jax.experimental.pallas import tpu as pltpu
from jax.experimental.pallas import tpu_sc as plsc

N, T, C, H, W = 8, 8, 128, 32, 32
F = C * H * W
TO = 16
NC, NS = 2, 16
NW = NC * NS
WPN = NW // N
CH = 4096
ITEMS = F // CH // WPN
UNROLL = 8
LANES = 16
NEG = -1e30


@functools.partial(
    pl.kernel,
    out_type=jax.ShapeDtypeStruct((N * TO * F,), jnp.float32),
    mesh=plsc.VectorSubcoreMesh(core_axis_name="c", subcore_axis_name="s"),
    compiler_params=pltpu.CompilerParams(needs_layout_passes=False),
    scratch_types=[
        pltpu.VMEM((2, T * CH), jnp.float32),
        pltpu.VMEM((4, CH), jnp.float32),
        pltpu.VMEM((2 * LANES,), jnp.float32),
        pltpu.SemaphoreType.DMA,
        pltpu.SemaphoreType.DMA,
        pltpu.SemaphoreType.DMA,
        pltpu.SemaphoreType.DMA,
        pltpu.SemaphoreType.DMA,
        pltpu.SemaphoreType.DMA,
    ],
)
def _interp_sc(u_hbm, idx_hbm, out_hbm, in_v, out_v, idx_v,
               sem_i0, sem_i1, sem_o0, sem_o1, sem_o2, sem_o3):
    cid = lax.axis_index("c")
    sid = lax.axis_index("s")
    wid = sid * NC + cid
    n = wid // WPN
    slot = wid % WPN

    pltpu.sync_copy(idx_hbm.at[pl.ds(n * (2 * LANES), 2 * LANES)], idx_v)
    v1 = idx_v[pl.ds(0, LANES)]
    v2 = idx_v[pl.ds(LANES, LANES)]
    m = jnp.maximum(jnp.max(v1), jnp.max(v2))
    e1 = jnp.exp(v1 - m)
    e2 = jnp.exp(v2 - m)
    tot = jnp.sum(e1) + jnp.sum(e2)
    tf = (plsc.cumsum(e1) * float(T - 1)) / jnp.broadcast_to(tot, (LANES,))
    tl = jnp.minimum(tf.astype(jnp.int32), T - 2)
    alpha = tf - tl.astype(jnp.float32)
    wl = 1.0 - alpha
    wr = alpha
    base_ls = [tl[o] * CH for o in range(TO)]
    wl_ss = [wl[o] for o in range(TO)]
    wr_ss = [wr[o] for o in range(TO)]

    isems = (sem_i0, sem_i1)
    osems = (sem_o0, sem_o1, sem_o2, sem_o3)

    def fire(item, ib):
        cbase = (slot * ITEMS + item) * CH
        for t in range(T):
            pltpu.make_async_copy(
                u_hbm.at[pl.ds((n * T + t) * F + cbase, CH)],
                in_v.at[ib, pl.ds(t * CH, CH)],
                isems[ib]).start()

    def drain_in(ib):
        pltpu.make_async_copy(u_hbm.at[pl.ds(0, T * CH)], in_v.at[ib],
                              isems[ib]).wait()

    def wait_out(b):
        pltpu.make_async_copy(out_v.at[b], out_hbm.at[pl.ds(0, CH)],
                              osems[b]).wait()

    def do_item(item, ib):
        cbase = (slot * ITEMS + item) * CH
        for po in range(0, TO, 2):
            o0, o1 = po, po + 1
            b0, b1 = o0 % 4, o1 % 4

            @pl.when(item * TO + po >= 4)
            def _():
                wait_out(b0)
                wait_out(b1)

            bl0, bl1 = base_ls[o0], base_ls[o1]
            w0l, w0r = wl_ss[o0], wr_ss[o0]
            w1l, w1r = wl_ss[o1], wr_ss[o1]
            shared = bl0 == bl1

            @pl.when(shared)
            def _():
                def body(off):
                    a = in_v[ib, pl.ds(bl0 + off, LANES)]
                    c = in_v[ib, pl.ds(bl0 + CH + off, LANES)]
                    out_v[b0, pl.ds(off, LANES)] = w0l * a + w0r * c
                    out_v[b1, pl.ds(off, LANES)] = w1l * a + w1r * c
                plsc.parallel_loop(0, CH, LANES, unroll=UNROLL)(body)

            adjacent = bl1 == bl0 + CH

            @pl.when(adjacent)
            def _():
                def body(off):
                    a0 = in_v[ib, pl.ds(bl0 + off, LANES)]
                    c0 = in_v[ib, pl.ds(bl0 + CH + off, LANES)]
                    c1 = in_v[ib, pl.ds(bl0 + 2 * CH + off, LANES)]
                    out_v[b0, pl.ds(off, LANES)] = w0l * a0 + w0r * c0
                    out_v[b1, pl.ds(off, LANES)] = w1l * c0 + w1r * c1
                plsc.parallel_loop(0, CH, LANES, unroll=UNROLL)(body)

            @pl.when(jnp.logical_not(jnp.logical_or(shared, adjacent)))
            def _():
                def body(off):
                    a0 = in_v[ib, pl.ds(bl0 + off, LANES)]
                    c0 = in_v[ib, pl.ds(bl0 + CH + off, LANES)]
                    out_v[b0, pl.ds(off, LANES)] = w0l * a0 + w0r * c0
                    a1 = in_v[ib, pl.ds(bl1 + off, LANES)]
                    c1 = in_v[ib, pl.ds(bl1 + CH + off, LANES)]
                    out_v[b1, pl.ds(off, LANES)] = w1l * a1 + w1r * c1
                plsc.parallel_loop(0, CH, LANES, unroll=UNROLL)(body)

            pltpu.make_async_copy(
                out_v.at[b0],
                out_hbm.at[pl.ds((n * TO + o0) * F + cbase, CH)],
                osems[b0]).start()
            pltpu.make_async_copy(
                out_v.at[b1],
                out_hbm.at[pl.ds((n * TO + o1) * F + cbase, CH)],
                osems[b1]).start()

    fire(0, 0)

    def pair(k, carry):
        item_a = 2 * k
        drain_in(0)
        fire(item_a + 1, 1)
        do_item(item_a, 0)
        drain_in(1)

        @pl.when(item_a + 2 < ITEMS)
        def _():
            fire(item_a + 2, 0)

        do_item(item_a + 1, 1)
        return carry

    lax.fori_loop(0, ITEMS // 2, pair, 0)

    for b in range(4):
        wait_out(b)


def kernel(input, index):
    u_flat = jnp.transpose(input, (0, 1, 3, 4, 2)).reshape(-1)
    idx_pad = jnp.pad(index, ((0, 0), (0, 2 * LANES - index.shape[1])),
                      constant_values=NEG).reshape(-1)
    out = _interp_sc(u_flat, idx_pad)
    return jnp.transpose(out.reshape(N, TO, H, W, C), (0, 1, 4, 2, 3))
